# Initial kernel scaffold; baseline (speedup 1.0000x reference)
import jax
import jax.numpy as jnp
from jax.experimental import pallas as pl


def kernel(img, conv1_w, conv1_b, fc_w, fc_b, l1_b0_c1_w, l1_b0_c1_b, l1_b0_c2_w, l1_b0_c2_b, l1_b1_c1_w, l1_b1_c1_b, l1_b1_c2_w, l1_b1_c2_b, l2_b0_c1_w, l2_b0_c1_b, l2_b0_c2_w, l2_b0_c2_b, l2_b0_ds_w, l2_b0_ds_b, l2_b1_c1_w, l2_b1_c1_b, l2_b1_c2_w, l2_b1_c2_b, l3_b0_c1_w, l3_b0_c1_b, l3_b0_c2_w, l3_b0_c2_b, l3_b0_ds_w, l3_b0_ds_b, l3_b1_c1_w, l3_b1_c1_b, l3_b1_c2_w, l3_b1_c2_b, l4_b0_c1_w, l4_b0_c1_b, l4_b0_c2_w, l4_b0_c2_b, l4_b0_ds_w, l4_b0_ds_b, l4_b1_c1_w, l4_b1_c1_b, l4_b1_c2_w, l4_b1_c2_b):
    raise NotImplementedError("write your pallas kernel here")


# speedup vs baseline: 1.1801x; 1.1801x over previous
import functools

import jax
import jax.numpy as jnp
from jax.experimental import pallas as pl
from jax.experimental.pallas import tpu as pltpu

_VMEM_LIMIT = 32 * 1024 * 1024
_MC = 1024


def _chunks(M):
    out = []
    m0 = 0
    while m0 < M:
        out.append((m0, min(_MC, M - m0)))
        m0 += _MC
    return out


def _mm_relu_kernel(a_ref, w_ref, b_ref, o_ref):
    acc = jnp.dot(a_ref[...], w_ref[...], preferred_element_type=jnp.float32)
    acc = jnp.maximum(acc + b_ref[...], 0.0)
    o_ref[...] = acc.astype(o_ref.dtype)


def _maxpool_kernel(ph_ref, mk_ref, o_ref, *, W2p, M, Sp):
    OC = o_ref.shape[-1]
    off = W2p + 1
    taps = tuple((2 * (di % 2) + (dj % 2), (di // 2) * W2p + (dj // 2))
                 for di in range(3) for dj in range(3))
    o_ref[0, 0:off, :] = jnp.zeros((off, OC), jnp.bfloat16)
    o_ref[0, off + M:Sp, :] = jnp.zeros((Sp - off - M, OC), jnp.bfloat16)
    for m0, mc in _chunks(M):
        r = None
        for p, toff in taps:
            v = ph_ref[0, p, pl.ds(m0 + toff, mc), :].astype(jnp.float32)
            r = v if r is None else jnp.maximum(r, v)
        r = r * mk_ref[pl.ds(m0, mc), :]
        o_ref[0, pl.ds(off + m0, mc), :] = r.astype(jnp.bfloat16)


def _blk_s1_kernel(x_ref, w1_ref, b1_ref, w2_ref, b2_ref, mk_ref, o_ref, scr,
                   *, Wp, M, Sp):
    OC = o_ref.shape[-1]
    off = Wp + 1
    taps = tuple(di * Wp + dj for di in range(3) for dj in range(3))
    scr[0:off, :] = jnp.zeros((off, OC), jnp.bfloat16)
    scr[off + M:Sp, :] = jnp.zeros((Sp - off - M, OC), jnp.bfloat16)
    for m0, mc in _chunks(M):
        acc = jnp.zeros((mc, OC), jnp.float32)
        for t, toff in enumerate(taps):
            acc = acc + jnp.dot(x_ref[0, pl.ds(m0 + toff, mc), :], w1_ref[t],
                                preferred_element_type=jnp.float32)
        acc = jnp.maximum(acc + b1_ref[...], 0.0) * mk_ref[pl.ds(m0, mc), :]
        scr[pl.ds(off + m0, mc), :] = acc.astype(jnp.bfloat16)
    o_ref[0, 0:off, :] = jnp.zeros((off, OC), jnp.bfloat16)
    o_ref[0, off + M:Sp, :] = jnp.zeros((Sp - off - M, OC), jnp.bfloat16)
    for m0, mc in _chunks(M):
        acc = jnp.zeros((mc, OC), jnp.float32)
        for t, toff in enumerate(taps):
            acc = acc + jnp.dot(scr[pl.ds(m0 + toff, mc), :], w2_ref[t],
                                preferred_element_type=jnp.float32)
        acc = acc + b2_ref[...] + x_ref[0, pl.ds(off + m0, mc), :].astype(jnp.float32)
        acc = jnp.maximum(acc, 0.0) * mk_ref[pl.ds(m0, mc), :]
        o_ref[0, pl.ds(off + m0, mc), :] = acc.astype(jnp.bfloat16)


def _blk_s2_kernel(ph_ref, w1_ref, b1_ref, wd_ref, bd_ref, w2_ref, b2_ref,
                   mk1_ref, mk2_ref, o_ref, scr, *, W2p, M, Sp):
    OC = o_ref.shape[-1]
    off = W2p + 1
    taps1 = tuple((2 * (di % 2) + (dj % 2), (di // 2) * W2p + (dj // 2))
                  for di in range(3) for dj in range(3))
    taps2 = tuple(di * W2p + dj for di in range(3) for dj in range(3))
    scr[0:off, :] = jnp.zeros((off, OC), jnp.bfloat16)
    scr[off + M:Sp, :] = jnp.zeros((Sp - off - M, OC), jnp.bfloat16)
    for m0, mc in _chunks(M):
        acc = jnp.zeros((mc, OC), jnp.float32)
        for t, (p, toff) in enumerate(taps1):
            acc = acc + jnp.dot(ph_ref[0, p, pl.ds(m0 + toff, mc), :], w1_ref[t],
                                preferred_element_type=jnp.float32)
        acc = jnp.maximum(acc + b1_ref[...], 0.0) * mk1_ref[pl.ds(m0, mc), :]
        scr[pl.ds(off + m0, mc), :] = acc.astype(jnp.bfloat16)
    o_ref[0, 0:off, :] = jnp.zeros((off, OC), jnp.bfloat16)
    o_ref[0, off + M:Sp, :] = jnp.zeros((Sp - off - M, OC), jnp.bfloat16)
    for m0, mc in _chunks(M):
        res = jnp.dot(ph_ref[0, 3, pl.ds(m0, mc), :], wd_ref[...],
                      preferred_element_type=jnp.float32) + bd_ref[...]
        res = res.astype(jnp.bfloat16).astype(jnp.float32)
        acc = jnp.zeros((mc, OC), jnp.float32)
        for t, toff in enumerate(taps2):
            acc = acc + jnp.dot(scr[pl.ds(m0 + toff, mc), :], w2_ref[t],
                                preferred_element_type=jnp.float32)
        acc = jnp.maximum(acc + b2_ref[...] + res, 0.0) * mk2_ref[pl.ds(m0, mc), :]
        o_ref[0, pl.ds(off + m0, mc), :] = acc.astype(jnp.bfloat16)


def _head_kernel(x_ref, w_ref, b_ref, o_ref, *, inv_cnt):
    s = jnp.sum(x_ref[...].astype(jnp.float32), axis=1) * inv_cnt
    o_ref[...] = jnp.dot(s.astype(jnp.bfloat16), w_ref[...],
                         preferred_element_type=jnp.float32) + b_ref[...]


def _cp(n_par):
    return pltpu.CompilerParams(dimension_semantics=("parallel",) * n_par,
                                vmem_limit_bytes=_VMEM_LIMIT)


def _interior_mask(H, W, Wp, M):
    q = jnp.arange(M, dtype=jnp.int32) + (Wp + 1)
    r, c = q // Wp, q % Wp
    ok = (r >= 1) & (r <= H) & (c >= 1) & (c <= W)
    return ok.astype(jnp.float32).reshape(M, 1)


def _col_mask(W2, W2p, M):
    ok = (jnp.arange(M, dtype=jnp.int32) % W2p) < W2
    return ok.astype(jnp.float32).reshape(M, 1)


def _stem_matmul(a, w, b):
    Mtot, K = a.shape
    OC = w.shape[-1]
    tm = 1024
    return pl.pallas_call(
        _mm_relu_kernel,
        out_shape=jax.ShapeDtypeStruct((Mtot, OC), jnp.bfloat16),
        grid=(Mtot // tm,),
        in_specs=[pl.BlockSpec((tm, K), lambda i: (i, 0)),
                  pl.BlockSpec((K, OC), lambda i: (0, 0)),
                  pl.BlockSpec((1, OC), lambda i: (0, 0))],
        out_specs=pl.BlockSpec((tm, OC), lambda i: (i, 0)),
        compiler_params=_cp(1),
    )(a, w, b.reshape(1, OC))


def _phases_nhwc(x4):
    N, Hp, Wp, C = x4.shape
    xp = jnp.pad(x4, ((0, 0), (0, 2), (0, 2), (0, 0)))
    ph = jnp.stack([xp[:, a::2, b::2, :] for a in (0, 1) for b in (0, 1)],
                   axis=1)
    H2p, W2p = ph.shape[2], ph.shape[3]
    return ph.reshape(N, 4, H2p * W2p, C), W2p


def _run_maxpool(ph, W2p, H2, W2):
    N, _, Sph, C = ph.shape
    M = H2 * W2p
    Sp = (H2 + 2) * W2p + 8
    mk = _interior_mask(H2, W2, W2p, M)
    return pl.pallas_call(
        functools.partial(_maxpool_kernel, W2p=W2p, M=M, Sp=Sp),
        out_shape=jax.ShapeDtypeStruct((N, Sp, C), jnp.bfloat16),
        grid=(N,),
        in_specs=[pl.BlockSpec((1, 4, Sph, C), lambda n: (n, 0, 0, 0)),
                  pl.BlockSpec((M, 1), lambda n: (0, 0))],
        out_specs=pl.BlockSpec((1, Sp, C), lambda n: (n, 0, 0)),
        compiler_params=_cp(1),
    )(ph, mk)


def _run_blk_s1(x, w1, b1, w2, b2, H, W):
    N, Sp, C = x.shape
    Wp = W + 2
    M = H * Wp
    OC = w1.shape[-1]
    mk = _interior_mask(H, W, Wp, M)
    return pl.pallas_call(
        functools.partial(_blk_s1_kernel, Wp=Wp, M=M, Sp=Sp),
        out_shape=jax.ShapeDtypeStruct((N, Sp, OC), jnp.bfloat16),
        grid=(N,),
        in_specs=[pl.BlockSpec((1, Sp, C), lambda n: (n, 0, 0)),
                  pl.BlockSpec(w1.shape, lambda n: (0, 0, 0)),
                  pl.BlockSpec((1, OC), lambda n: (0, 0)),
                  pl.BlockSpec(w2.shape, lambda n: (0, 0, 0)),
                  pl.BlockSpec((1, OC), lambda n: (0, 0)),
                  pl.BlockSpec((M, 1), lambda n: (0, 0))],
        out_specs=pl.BlockSpec((1, Sp, OC), lambda n: (n, 0, 0)),
        scratch_shapes=[pltpu.VMEM((Sp, OC), jnp.bfloat16)],
        compiler_params=_cp(1),
    )(x, w1, b1.reshape(1, OC), w2, b2.reshape(1, OC), mk)


def _run_blk_s2(x, w1, b1, wd, bd, w2, b2, H, W):
    N, _, C = x.shape
    Hp, Wp = H + 2, W + 2
    x4 = x[:, :Hp * Wp, :].reshape(N, Hp, Wp, C)
    ph, W2p = _phases_nhwc(x4)
    Sph = ph.shape[2]
    H2, W2 = H // 2, W // 2
    M = H2 * W2p
    Sp = (H2 + 2) * W2p + 8
    OC = w1.shape[-1]
    mk1 = _col_mask(W2, W2p, M)
    mk2 = _interior_mask(H2, W2, W2p, M)
    return pl.pallas_call(
        functools.partial(_blk_s2_kernel, W2p=W2p, M=M, Sp=Sp),
        out_shape=jax.ShapeDtypeStruct((N, Sp, OC), jnp.bfloat16),
        grid=(N,),
        in_specs=[pl.BlockSpec((1, 4, Sph, C), lambda n: (n, 0, 0, 0)),
                  pl.BlockSpec(w1.shape, lambda n: (0, 0, 0)),
                  pl.BlockSpec((1, OC), lambda n: (0, 0)),
                  pl.BlockSpec(wd.shape, lambda n: (0, 0)),
                  pl.BlockSpec((1, OC), lambda n: (0, 0)),
                  pl.BlockSpec(w2.shape, lambda n: (0, 0, 0)),
                  pl.BlockSpec((1, OC), lambda n: (0, 0)),
                  pl.BlockSpec((M, 1), lambda n: (0, 0)),
                  pl.BlockSpec((M, 1), lambda n: (0, 0))],
        out_specs=pl.BlockSpec((1, Sp, OC), lambda n: (n, 0, 0)),
        scratch_shapes=[pltpu.VMEM((Sp, OC), jnp.bfloat16)],
        compiler_params=_cp(1),
    )(ph, w1, b1.reshape(1, OC), wd, bd.reshape(1, OC), w2, b2.reshape(1, OC),
      mk1, mk2)


def _run_head(x, w, b, cnt):
    N, Sp, C = x.shape
    D = w.shape[-1]
    return pl.pallas_call(
        functools.partial(_head_kernel, inv_cnt=1.0 / cnt),
        out_shape=jax.ShapeDtypeStruct((N, D), jnp.float32),
        grid=(1,),
        in_specs=[pl.BlockSpec((N, Sp, C), lambda i: (0, 0, 0)),
                  pl.BlockSpec((C, D), lambda i: (0, 0)),
                  pl.BlockSpec((1, D), lambda i: (0, 0))],
        out_specs=pl.BlockSpec((N, D), lambda i: (0, 0)),
        compiler_params=pltpu.CompilerParams(
            dimension_semantics=("arbitrary",),
            vmem_limit_bytes=_VMEM_LIMIT),
    )(x, w, b.reshape(1, D))


def kernel(img, conv1_w, conv1_b, fc_w, fc_b,
           l1_b0_c1_w, l1_b0_c1_b, l1_b0_c2_w, l1_b0_c2_b,
           l1_b1_c1_w, l1_b1_c1_b, l1_b1_c2_w, l1_b1_c2_b,
           l2_b0_c1_w, l2_b0_c1_b, l2_b0_c2_w, l2_b0_c2_b, l2_b0_ds_w, l2_b0_ds_b,
           l2_b1_c1_w, l2_b1_c1_b, l2_b1_c2_w, l2_b1_c2_b,
           l3_b0_c1_w, l3_b0_c1_b, l3_b0_c2_w, l3_b0_c2_b, l3_b0_ds_w, l3_b0_ds_b,
           l3_b1_c1_w, l3_b1_c1_b, l3_b1_c2_w, l3_b1_c2_b,
           l4_b0_c1_w, l4_b0_c1_b, l4_b0_c2_w, l4_b0_c2_b, l4_b0_ds_w, l4_b0_ds_b,
           l4_b1_c1_w, l4_b1_c1_b, l4_b1_c2_w, l4_b1_c2_b):
    N, _, HI, WI = img.shape
    oh, ow = HI // 2, WI // 2
    H1 = oh // 2
    x = jnp.transpose(img, (0, 2, 3, 1)).astype(jnp.bfloat16)

    xp = jnp.pad(x, ((0, 0), (3, 3), (3, 3), (0, 0)))
    cols = [xp[:, i:i + 2 * oh:2, j:j + 2 * ow:2, :]
            for i in range(7) for j in range(7)]
    patches = jnp.stack(cols, axis=3).reshape(N * oh * ow, 7 * 7 * 3)
    stem = _stem_matmul(patches, conv1_w, conv1_b).reshape(N, oh, ow, 64)

    ph, W2p = _phases_nhwc(jnp.pad(stem, ((0, 0), (1, 1), (1, 1), (0, 0))))
    x1 = _run_maxpool(ph, W2p, H1, H1)

    x1 = _run_blk_s1(x1, l1_b0_c1_w, l1_b0_c1_b, l1_b0_c2_w, l1_b0_c2_b, H1, H1)
    x1 = _run_blk_s1(x1, l1_b1_c1_w, l1_b1_c1_b, l1_b1_c2_w, l1_b1_c2_b, H1, H1)

    H2 = H1 // 2
    x2 = _run_blk_s2(x1, l2_b0_c1_w, l2_b0_c1_b, l2_b0_ds_w, l2_b0_ds_b,
                     l2_b0_c2_w, l2_b0_c2_b, H1, H1)
    x2 = _run_blk_s1(x2, l2_b1_c1_w, l2_b1_c1_b, l2_b1_c2_w, l2_b1_c2_b, H2, H2)

    H3 = H2 // 2
    x3 = _run_blk_s2(x2, l3_b0_c1_w, l3_b0_c1_b, l3_b0_ds_w, l3_b0_ds_b,
                     l3_b0_c2_w, l3_b0_c2_b, H2, H2)
    x3 = _run_blk_s1(x3, l3_b1_c1_w, l3_b1_c1_b, l3_b1_c2_w, l3_b1_c2_b, H3, H3)

    H4 = H3 // 2
    x4 = _run_blk_s2(x3, l4_b0_c1_w, l4_b0_c1_b, l4_b0_ds_w, l4_b0_ds_b,
                     l4_b0_c2_w, l4_b0_c2_b, H3, H3)
    x4 = _run_blk_s1(x4, l4_b1_c1_w, l4_b1_c1_b, l4_b1_c2_w, l4_b1_c2_b, H4, H4)

    logits = _run_head(x4, fc_w, fc_b, float(H4 * H4))
    return logits, None, None


# speedup vs baseline: 1.1897x; 1.0081x over previous
import functools

import jax
import jax.numpy as jnp
from jax.experimental import pallas as pl
from jax.experimental.pallas import tpu as pltpu

_VMEM_LIMIT = 32 * 1024 * 1024
_MC = 1024


def _chunks(M):
    out = []
    m0 = 0
    while m0 < M:
        out.append((m0, min(_MC, M - m0)))
        m0 += _MC
    return out


def _mm_relu_kernel(a_ref, w_ref, b_ref, o_ref):
    acc = jnp.dot(a_ref[...], w_ref[...], preferred_element_type=jnp.float32)
    acc = jnp.maximum(acc + b_ref[...], 0.0)
    o_ref[...] = acc.astype(o_ref.dtype)


def _maxpool_kernel(ph_ref, mk_ref, o_ref, *, W2p, M, Sp):
    OC = o_ref.shape[-1]
    off = W2p + 1
    taps = tuple((2 * (di % 2) + (dj % 2), (di // 2) * W2p + (dj // 2))
                 for di in range(3) for dj in range(3))
    o_ref[0, 0:off, :] = jnp.zeros((off, OC), jnp.bfloat16)
    o_ref[0, off + M:Sp, :] = jnp.zeros((Sp - off - M, OC), jnp.bfloat16)
    for m0, mc in _chunks(M):
        r = None
        for p, toff in taps:
            v = ph_ref[0, p, pl.ds(m0 + toff, mc), :].astype(jnp.float32)
            r = v if r is None else jnp.maximum(r, v)
        r = r * mk_ref[pl.ds(m0, mc), :]
        o_ref[0, pl.ds(off + m0, mc), :] = r.astype(jnp.bfloat16)


def _tap_mm(slices, w_ref):
    if len(w_ref.shape) == 2:
        a = jnp.concatenate(slices, axis=1)
        return jnp.dot(a, w_ref[...], preferred_element_type=jnp.float32)
    acc = jnp.dot(slices[0], w_ref[0], preferred_element_type=jnp.float32)
    for t in range(1, len(slices)):
        acc = acc + jnp.dot(slices[t], w_ref[t],
                            preferred_element_type=jnp.float32)
    return acc


def _blk_s1_kernel(x_ref, w1_ref, b1_ref, w2_ref, b2_ref, mk_ref, o_ref, scr,
                   *, Wp, M, Sp, G):
    OC = o_ref.shape[-1]
    off = Wp + 1
    taps = tuple(di * Wp + dj for di in range(3) for dj in range(3))
    for g in range(G):
        sb = g * Sp
        scr[sb:sb + off, :] = jnp.zeros((off, OC), jnp.bfloat16)
        scr[sb + off + M:sb + Sp, :] = jnp.zeros((Sp - off - M, OC), jnp.bfloat16)
        for m0, mc in _chunks(M):
            acc = _tap_mm([x_ref[g, pl.ds(m0 + toff, mc), :] for toff in taps],
                          w1_ref)
            acc = jnp.maximum(acc + b1_ref[...], 0.0) * mk_ref[pl.ds(m0, mc), :]
            scr[pl.ds(sb + off + m0, mc), :] = acc.astype(jnp.bfloat16)
        o_ref[g, 0:off, :] = jnp.zeros((off, OC), jnp.bfloat16)
        o_ref[g, off + M:Sp, :] = jnp.zeros((Sp - off - M, OC), jnp.bfloat16)
        for m0, mc in _chunks(M):
            acc = _tap_mm([scr[pl.ds(sb + m0 + toff, mc), :] for toff in taps],
                          w2_ref)
            acc = acc + b2_ref[...] + x_ref[g, pl.ds(off + m0, mc), :].astype(jnp.float32)
            acc = jnp.maximum(acc, 0.0) * mk_ref[pl.ds(m0, mc), :]
            o_ref[g, pl.ds(off + m0, mc), :] = acc.astype(jnp.bfloat16)


def _blk_s2_kernel(ph_ref, w1_ref, b1_ref, wd_ref, bd_ref, w2_ref, b2_ref,
                   mk1_ref, mk2_ref, o_ref, scr, *, W2p, M, Sp, G):
    OC = o_ref.shape[-1]
    off = W2p + 1
    taps1 = tuple((2 * (di % 2) + (dj % 2), (di // 2) * W2p + (dj // 2))
                  for di in range(3) for dj in range(3))
    taps2 = tuple(di * W2p + dj for di in range(3) for dj in range(3))
    for g in range(G):
        sb = g * Sp
        scr[sb:sb + off, :] = jnp.zeros((off, OC), jnp.bfloat16)
        scr[sb + off + M:sb + Sp, :] = jnp.zeros((Sp - off - M, OC), jnp.bfloat16)
        for m0, mc in _chunks(M):
            acc = _tap_mm([ph_ref[g, p, pl.ds(m0 + toff, mc), :]
                           for p, toff in taps1], w1_ref)
            acc = jnp.maximum(acc + b1_ref[...], 0.0) * mk1_ref[pl.ds(m0, mc), :]
            scr[pl.ds(sb + off + m0, mc), :] = acc.astype(jnp.bfloat16)
        o_ref[g, 0:off, :] = jnp.zeros((off, OC), jnp.bfloat16)
        o_ref[g, off + M:Sp, :] = jnp.zeros((Sp - off - M, OC), jnp.bfloat16)
        for m0, mc in _chunks(M):
            res = jnp.dot(ph_ref[g, 3, pl.ds(m0, mc), :], wd_ref[...],
                          preferred_element_type=jnp.float32) + bd_ref[...]
            res = res.astype(jnp.bfloat16).astype(jnp.float32)
            acc = _tap_mm([scr[pl.ds(sb + m0 + toff, mc), :] for toff in taps2],
                          w2_ref)
            acc = acc + b2_ref[...] + res
            acc = jnp.maximum(acc, 0.0) * mk2_ref[pl.ds(m0, mc), :]
            o_ref[g, pl.ds(off + m0, mc), :] = acc.astype(jnp.bfloat16)


def _head_kernel(x_ref, w_ref, b_ref, o_ref, *, inv_cnt):
    s = jnp.sum(x_ref[...].astype(jnp.float32), axis=1) * inv_cnt
    o_ref[...] = jnp.dot(s.astype(jnp.bfloat16), w_ref[...],
                         preferred_element_type=jnp.float32) + b_ref[...]


def _cp(n_par):
    return pltpu.CompilerParams(dimension_semantics=("parallel",) * n_par,
                                vmem_limit_bytes=_VMEM_LIMIT)


def _interior_mask(H, W, Wp, M):
    q = jnp.arange(M, dtype=jnp.int32) + (Wp + 1)
    r, c = q // Wp, q % Wp
    ok = (r >= 1) & (r <= H) & (c >= 1) & (c <= W)
    return ok.astype(jnp.float32).reshape(M, 1)


def _col_mask(W2, W2p, M):
    ok = (jnp.arange(M, dtype=jnp.int32) % W2p) < W2
    return ok.astype(jnp.float32).reshape(M, 1)


def _stem_matmul(a, w, b):
    Mtot, K = a.shape
    OC = w.shape[-1]
    tm = 1024
    return pl.pallas_call(
        _mm_relu_kernel,
        out_shape=jax.ShapeDtypeStruct((Mtot, OC), jnp.bfloat16),
        grid=(Mtot // tm,),
        in_specs=[pl.BlockSpec((tm, K), lambda i: (i, 0)),
                  pl.BlockSpec((K, OC), lambda i: (0, 0)),
                  pl.BlockSpec((1, OC), lambda i: (0, 0))],
        out_specs=pl.BlockSpec((tm, OC), lambda i: (i, 0)),
        compiler_params=_cp(1),
    )(a, w, b.reshape(1, OC))


def _phases_nhwc(x4):
    N, Hp, Wp, C = x4.shape
    xp = jnp.pad(x4, ((0, 0), (0, 2), (0, 2), (0, 0)))
    ph = jnp.stack([xp[:, a::2, b::2, :] for a in (0, 1) for b in (0, 1)],
                   axis=1)
    H2p, W2p = ph.shape[2], ph.shape[3]
    return ph.reshape(N, 4, H2p * W2p, C), W2p


def _run_maxpool(ph, W2p, H2, W2):
    N, _, Sph, C = ph.shape
    M = H2 * W2p
    Sp = (H2 + 2) * W2p + 8
    mk = _interior_mask(H2, W2, W2p, M)
    return pl.pallas_call(
        functools.partial(_maxpool_kernel, W2p=W2p, M=M, Sp=Sp),
        out_shape=jax.ShapeDtypeStruct((N, Sp, C), jnp.bfloat16),
        grid=(N,),
        in_specs=[pl.BlockSpec((1, 4, Sph, C), lambda n: (n, 0, 0, 0)),
                  pl.BlockSpec((M, 1), lambda n: (0, 0))],
        out_specs=pl.BlockSpec((1, Sp, C), lambda n: (n, 0, 0)),
        compiler_params=_cp(1),
    )(ph, mk)


def _cat_w(w):
    T, IC, OC = w.shape
    return w.reshape(T * IC, OC) if IC < 256 else w


def _wspec(w):
    return pl.BlockSpec(w.shape, (lambda n: (0, 0)) if w.ndim == 2
                        else (lambda n: (0, 0, 0)))


def _run_blk_s1(x, w1, b1, w2, b2, H, W, G=1):
    N, Sp, C = x.shape
    Wp = W + 2
    M = H * Wp
    OC = w1.shape[-1]
    w1, w2 = _cat_w(w1), _cat_w(w2)
    mk = _interior_mask(H, W, Wp, M)
    return pl.pallas_call(
        functools.partial(_blk_s1_kernel, Wp=Wp, M=M, Sp=Sp, G=G),
        out_shape=jax.ShapeDtypeStruct((N, Sp, OC), jnp.bfloat16),
        grid=(N // G,),
        in_specs=[pl.BlockSpec((G, Sp, C), lambda n: (n, 0, 0)),
                  _wspec(w1),
                  pl.BlockSpec((1, OC), lambda n: (0, 0)),
                  _wspec(w2),
                  pl.BlockSpec((1, OC), lambda n: (0, 0)),
                  pl.BlockSpec((M, 1), lambda n: (0, 0))],
        out_specs=pl.BlockSpec((G, Sp, OC), lambda n: (n, 0, 0)),
        scratch_shapes=[pltpu.VMEM((G * Sp, OC), jnp.bfloat16)],
        compiler_params=_cp(1),
    )(x, w1, b1.reshape(1, OC), w2, b2.reshape(1, OC), mk)


def _run_blk_s2(x, w1, b1, wd, bd, w2, b2, H, W, G=1):
    N, _, C = x.shape
    Hp, Wp = H + 2, W + 2
    x4 = x[:, :Hp * Wp, :].reshape(N, Hp, Wp, C)
    ph, W2p = _phases_nhwc(x4)
    Sph = ph.shape[2]
    H2, W2 = H // 2, W // 2
    M = H2 * W2p
    Sp = (H2 + 2) * W2p + 8
    OC = w1.shape[-1]
    w1, w2 = _cat_w(w1), _cat_w(w2)
    mk1 = _col_mask(W2, W2p, M)
    mk2 = _interior_mask(H2, W2, W2p, M)
    return pl.pallas_call(
        functools.partial(_blk_s2_kernel, W2p=W2p, M=M, Sp=Sp, G=G),
        out_shape=jax.ShapeDtypeStruct((N, Sp, OC), jnp.bfloat16),
        grid=(N // G,),
        in_specs=[pl.BlockSpec((G, 4, Sph, C), lambda n: (n, 0, 0, 0)),
                  _wspec(w1),
                  pl.BlockSpec((1, OC), lambda n: (0, 0)),
                  pl.BlockSpec(wd.shape, lambda n: (0, 0)),
                  pl.BlockSpec((1, OC), lambda n: (0, 0)),
                  _wspec(w2),
                  pl.BlockSpec((1, OC), lambda n: (0, 0)),
                  pl.BlockSpec((M, 1), lambda n: (0, 0)),
                  pl.BlockSpec((M, 1), lambda n: (0, 0))],
        out_specs=pl.BlockSpec((G, Sp, OC), lambda n: (n, 0, 0)),
        scratch_shapes=[pltpu.VMEM((G * Sp, OC), jnp.bfloat16)],
        compiler_params=_cp(1),
    )(ph, w1, b1.reshape(1, OC), wd, bd.reshape(1, OC), w2, b2.reshape(1, OC),
      mk1, mk2)


def _run_head(x, w, b, cnt):
    N, Sp, C = x.shape
    D = w.shape[-1]
    return pl.pallas_call(
        functools.partial(_head_kernel, inv_cnt=1.0 / cnt),
        out_shape=jax.ShapeDtypeStruct((N, D), jnp.float32),
        grid=(1,),
        in_specs=[pl.BlockSpec((N, Sp, C), lambda i: (0, 0, 0)),
                  pl.BlockSpec((C, D), lambda i: (0, 0)),
                  pl.BlockSpec((1, D), lambda i: (0, 0))],
        out_specs=pl.BlockSpec((N, D), lambda i: (0, 0)),
        compiler_params=pltpu.CompilerParams(
            dimension_semantics=("arbitrary",),
            vmem_limit_bytes=_VMEM_LIMIT),
    )(x, w, b.reshape(1, D))


def kernel(img, conv1_w, conv1_b, fc_w, fc_b,
           l1_b0_c1_w, l1_b0_c1_b, l1_b0_c2_w, l1_b0_c2_b,
           l1_b1_c1_w, l1_b1_c1_b, l1_b1_c2_w, l1_b1_c2_b,
           l2_b0_c1_w, l2_b0_c1_b, l2_b0_c2_w, l2_b0_c2_b, l2_b0_ds_w, l2_b0_ds_b,
           l2_b1_c1_w, l2_b1_c1_b, l2_b1_c2_w, l2_b1_c2_b,
           l3_b0_c1_w, l3_b0_c1_b, l3_b0_c2_w, l3_b0_c2_b, l3_b0_ds_w, l3_b0_ds_b,
           l3_b1_c1_w, l3_b1_c1_b, l3_b1_c2_w, l3_b1_c2_b,
           l4_b0_c1_w, l4_b0_c1_b, l4_b0_c2_w, l4_b0_c2_b, l4_b0_ds_w, l4_b0_ds_b,
           l4_b1_c1_w, l4_b1_c1_b, l4_b1_c2_w, l4_b1_c2_b):
    N, _, HI, WI = img.shape
    oh, ow = HI // 2, WI // 2
    H1 = oh // 2
    x = jnp.transpose(img, (0, 2, 3, 1)).astype(jnp.bfloat16)

    xp = jnp.pad(x, ((0, 0), (3, 3), (3, 3), (0, 0)))
    cols = [xp[:, i:i + 2 * oh:2, j:j + 2 * ow:2, :]
            for i in range(7) for j in range(7)]
    patches = jnp.stack(cols, axis=3).reshape(N * oh * ow, 7 * 7 * 3)
    stem = _stem_matmul(patches, conv1_w, conv1_b).reshape(N, oh, ow, 64)

    ph, W2p = _phases_nhwc(jnp.pad(stem, ((0, 0), (1, 1), (1, 1), (0, 0))))
    x1 = _run_maxpool(ph, W2p, H1, H1)

    x1 = _run_blk_s1(x1, l1_b0_c1_w, l1_b0_c1_b, l1_b0_c2_w, l1_b0_c2_b, H1, H1)
    x1 = _run_blk_s1(x1, l1_b1_c1_w, l1_b1_c1_b, l1_b1_c2_w, l1_b1_c2_b, H1, H1)

    H2 = H1 // 2
    x2 = _run_blk_s2(x1, l2_b0_c1_w, l2_b0_c1_b, l2_b0_ds_w, l2_b0_ds_b,
                     l2_b0_c2_w, l2_b0_c2_b, H1, H1)
    x2 = _run_blk_s1(x2, l2_b1_c1_w, l2_b1_c1_b, l2_b1_c2_w, l2_b1_c2_b, H2, H2)

    H3 = H2 // 2
    x3 = _run_blk_s2(x2, l3_b0_c1_w, l3_b0_c1_b, l3_b0_ds_w, l3_b0_ds_b,
                     l3_b0_c2_w, l3_b0_c2_b, H2, H2)
    x3 = _run_blk_s1(x3, l3_b1_c1_w, l3_b1_c1_b, l3_b1_c2_w, l3_b1_c2_b, H3, H3)

    H4 = H3 // 2
    G4 = 4 if N % 4 == 0 else 1
    x4 = _run_blk_s2(x3, l4_b0_c1_w, l4_b0_c1_b, l4_b0_ds_w, l4_b0_ds_b,
                     l4_b0_c2_w, l4_b0_c2_b, H3, H3, G=G4)
    x4 = _run_blk_s1(x4, l4_b1_c1_w, l4_b1_c1_b, l4_b1_c2_w, l4_b1_c2_b, H4, H4,
                     G=G4)

    logits = _run_head(x4, fc_w, fc_b, float(H4 * H4))
    return logits, None, None


# speedup vs baseline: 1.7274x; 1.4520x over previous
import functools

import jax
import jax.numpy as jnp
from jax.experimental import pallas as pl
from jax.experimental.pallas import tpu as pltpu

_VMEM_LIMIT = 32 * 1024 * 1024
_MC = 1024


def _chunks(M):
    out = []
    m0 = 0
    while m0 < M:
        out.append((m0, min(_MC, M - m0)))
        m0 += _MC
    return out


def _mm_relu_kernel(a_ref, w_ref, b_ref, o_ref):
    acc = jnp.dot(a_ref[...], w_ref[...], preferred_element_type=jnp.float32)
    acc = jnp.maximum(acc + b_ref[...], 0.0)
    o_ref[...] = acc.astype(o_ref.dtype)


def _maxpool_kernel(x_ref, mk_ref, o_ref, ph_scr, *, OH, OW, W2p, M, Sp):
    C = ph_scr.shape[-1]
    off = W2p + 1
    taps = tuple((2 * (di % 2) + (dj % 2), (di // 2) * W2p + (dj // 2))
                 for di in range(3) for dj in range(3))
    ph_scr[...] = jnp.zeros(ph_scr.shape, jnp.bfloat16)
    for a in (0, 1):
        for b in (0, 1):
            p = 2 * a + b
            j0 = 1 if b == 0 else 0
            c0 = 2 * j0 + b - 1
            cnt = (OW - 1 - c0) // 2 + 1
            lo = (c0 % 2) * C
            for i in range(OH // 2 + 2):
                r = 2 * i + a - 1
                if r < 0 or r >= OH:
                    continue
                k0 = (r * OW + c0) // 2
                ph_scr[p, i * W2p + j0:i * W2p + j0 + cnt, :] = \
                    x_ref[0, pl.ds(k0, cnt), lo:lo + C]
    o_ref[0, 0:off, :] = jnp.zeros((off, C), jnp.bfloat16)
    o_ref[0, off + M:Sp, :] = jnp.zeros((Sp - off - M, C), jnp.bfloat16)
    for m0, mc in _chunks(M):
        r = None
        for p, toff in taps:
            v = ph_scr[p, pl.ds(m0 + toff, mc), :].astype(jnp.float32)
            r = v if r is None else jnp.maximum(r, v)
        r = r * mk_ref[pl.ds(m0, mc), :]
        o_ref[0, pl.ds(off + m0, mc), :] = r.astype(jnp.bfloat16)


def _tap_mm(slices, w_ref):
    if len(w_ref.shape) == 2:
        a = jnp.concatenate(slices, axis=1)
        return jnp.dot(a, w_ref[...], preferred_element_type=jnp.float32)
    acc = jnp.dot(slices[0], w_ref[0], preferred_element_type=jnp.float32)
    for t in range(1, len(slices)):
        acc = acc + jnp.dot(slices[t], w_ref[t],
                            preferred_element_type=jnp.float32)
    return acc


def _blk_s1_kernel(x_ref, w1_ref, b1_ref, w2_ref, b2_ref, mk_ref, o_ref, scr,
                   *, Wp, M, Sp, G):
    OC = o_ref.shape[-1]
    off = Wp + 1
    taps = tuple(di * Wp + dj for di in range(3) for dj in range(3))
    for g in range(G):
        sb = g * Sp
        scr[sb:sb + off, :] = jnp.zeros((off, OC), jnp.bfloat16)
        scr[sb + off + M:sb + Sp, :] = jnp.zeros((Sp - off - M, OC), jnp.bfloat16)
        for m0, mc in _chunks(M):
            acc = _tap_mm([x_ref[g, pl.ds(m0 + toff, mc), :] for toff in taps],
                          w1_ref)
            acc = jnp.maximum(acc + b1_ref[...], 0.0) * mk_ref[pl.ds(m0, mc), :]
            scr[pl.ds(sb + off + m0, mc), :] = acc.astype(jnp.bfloat16)
        o_ref[g, 0:off, :] = jnp.zeros((off, OC), jnp.bfloat16)
        o_ref[g, off + M:Sp, :] = jnp.zeros((Sp - off - M, OC), jnp.bfloat16)
        for m0, mc in _chunks(M):
            acc = _tap_mm([scr[pl.ds(sb + m0 + toff, mc), :] for toff in taps],
                          w2_ref)
            acc = acc + b2_ref[...] + x_ref[g, pl.ds(off + m0, mc), :].astype(jnp.float32)
            acc = jnp.maximum(acc, 0.0) * mk_ref[pl.ds(m0, mc), :]
            o_ref[g, pl.ds(off + m0, mc), :] = acc.astype(jnp.bfloat16)


def _blk_s2_kernel(x_ref, w1_ref, b1_ref, wd_ref, bd_ref, w2_ref, b2_ref,
                   mk1_ref, mk2_ref, o_ref, ph_scr, scr,
                   *, Wp_in, H, W, W2p, M, Sp, G):
    OC = o_ref.shape[-1]
    off = W2p + 1
    H2 = H // 2
    cnt = W // 2 + 1
    taps1 = tuple((2 * (di % 2) + (dj % 2), (di // 2) * W2p + (dj // 2))
                  for di in range(3) for dj in range(3))
    taps2 = tuple(di * W2p + dj for di in range(3) for dj in range(3))
    C = ph_scr.shape[-1]
    for g in range(G):
        ph_scr[...] = jnp.zeros(ph_scr.shape, jnp.bfloat16)
        for a in (0, 1):
            for b in (0, 1):
                p = 2 * a + b
                lo = b * C
                for i in range(H2 + 2):
                    k0 = (2 * i + a) * (Wp_in // 2)
                    ph_scr[p, i * W2p:i * W2p + cnt, :] = \
                        x_ref[g, pl.ds(k0, cnt), lo:lo + C]
        scr[0:off, :] = jnp.zeros((off, OC), jnp.bfloat16)
        scr[off + M:Sp, :] = jnp.zeros((Sp - off - M, OC), jnp.bfloat16)
        for m0, mc in _chunks(M):
            acc = _tap_mm([ph_scr[p, pl.ds(m0 + toff, mc), :]
                           for p, toff in taps1], w1_ref)
            acc = jnp.maximum(acc + b1_ref[...], 0.0) * mk1_ref[pl.ds(m0, mc), :]
            scr[pl.ds(off + m0, mc), :] = acc.astype(jnp.bfloat16)
        o_ref[g, 0:off, :] = jnp.zeros((off, OC), jnp.bfloat16)
        o_ref[g, off + M:Sp, :] = jnp.zeros((Sp - off - M, OC), jnp.bfloat16)
        for m0, mc in _chunks(M):
            res = jnp.dot(ph_scr[3, pl.ds(m0, mc), :], wd_ref[...],
                          preferred_element_type=jnp.float32) + bd_ref[...]
            res = res.astype(jnp.bfloat16).astype(jnp.float32)
            acc = _tap_mm([scr[pl.ds(m0 + toff, mc), :] for toff in taps2],
                          w2_ref)
            acc = acc + b2_ref[...] + res
            acc = jnp.maximum(acc, 0.0) * mk2_ref[pl.ds(m0, mc), :]
            o_ref[g, pl.ds(off + m0, mc), :] = acc.astype(jnp.bfloat16)


def _head_kernel(x_ref, w_ref, b_ref, o_ref, *, inv_cnt):
    s = jnp.sum(x_ref[...].astype(jnp.float32), axis=1) * inv_cnt
    o_ref[...] = jnp.dot(s.astype(jnp.bfloat16), w_ref[...],
                         preferred_element_type=jnp.float32) + b_ref[...]


def _cp(n_par):
    return pltpu.CompilerParams(dimension_semantics=("parallel",) * n_par,
                                vmem_limit_bytes=_VMEM_LIMIT)


def _interior_mask(H, W, Wp, M):
    q = jnp.arange(M, dtype=jnp.int32) + (Wp + 1)
    r, c = q // Wp, q % Wp
    ok = (r >= 1) & (r <= H) & (c >= 1) & (c <= W)
    return ok.astype(jnp.float32).reshape(M, 1)


def _col_mask(W2, W2p, M):
    ok = (jnp.arange(M, dtype=jnp.int32) % W2p) < W2
    return ok.astype(jnp.float32).reshape(M, 1)


def _stem_matmul(a, w, b):
    Mtot, K = a.shape
    OC = w.shape[-1]
    tm = 1024
    return pl.pallas_call(
        _mm_relu_kernel,
        out_shape=jax.ShapeDtypeStruct((Mtot, OC), jnp.bfloat16),
        grid=(Mtot // tm,),
        in_specs=[pl.BlockSpec((tm, K), lambda i: (i, 0)),
                  pl.BlockSpec((K, OC), lambda i: (0, 0)),
                  pl.BlockSpec((1, OC), lambda i: (0, 0))],
        out_specs=pl.BlockSpec((tm, OC), lambda i: (i, 0)),
        compiler_params=_cp(1),
    )(a, w, b.reshape(1, OC))


def _run_maxpool(x, OH, OW):
    N, _, C = x.shape
    H2, W2 = OH // 2, OW // 2
    W2p = W2 + 2
    M = H2 * W2p
    Sp = (H2 + 4) * W2p
    mk = _interior_mask(H2, W2, W2p, M)
    return pl.pallas_call(
        functools.partial(_maxpool_kernel, OH=OH, OW=OW, W2p=W2p, M=M, Sp=Sp),
        out_shape=jax.ShapeDtypeStruct((N, Sp, C), jnp.bfloat16),
        grid=(N,),
        in_specs=[pl.BlockSpec((1, OH * OW // 2, 2 * C), lambda n: (n, 0, 0)),
                  pl.BlockSpec((M, 1), lambda n: (0, 0))],
        out_specs=pl.BlockSpec((1, Sp, C), lambda n: (n, 0, 0)),
        scratch_shapes=[pltpu.VMEM((4, (H2 + 2) * W2p, C), jnp.bfloat16)],
        compiler_params=_cp(1),
    )(x.reshape(N, OH * OW // 2, 2 * C), mk)


def _cat_w(w):
    T, IC, OC = w.shape
    return w.reshape(T * IC, OC) if IC < 256 else w


def _wspec(w):
    return pl.BlockSpec(w.shape, (lambda n: (0, 0)) if w.ndim == 2
                        else (lambda n: (0, 0, 0)))


def _run_blk_s1(x, w1, b1, w2, b2, H, W, G=1):
    N, Sp, C = x.shape
    Wp = W + 2
    M = H * Wp
    OC = w1.shape[-1]
    w1, w2 = _cat_w(w1), _cat_w(w2)
    mk = _interior_mask(H, W, Wp, M)
    return pl.pallas_call(
        functools.partial(_blk_s1_kernel, Wp=Wp, M=M, Sp=Sp, G=G),
        out_shape=jax.ShapeDtypeStruct((N, Sp, OC), jnp.bfloat16),
        grid=(N // G,),
        in_specs=[pl.BlockSpec((G, Sp, C), lambda n: (n, 0, 0)),
                  _wspec(w1),
                  pl.BlockSpec((1, OC), lambda n: (0, 0)),
                  _wspec(w2),
                  pl.BlockSpec((1, OC), lambda n: (0, 0)),
                  pl.BlockSpec((M, 1), lambda n: (0, 0))],
        out_specs=pl.BlockSpec((G, Sp, OC), lambda n: (n, 0, 0)),
        scratch_shapes=[pltpu.VMEM((G * Sp, OC), jnp.bfloat16)],
        compiler_params=_cp(1),
    )(x, w1, b1.reshape(1, OC), w2, b2.reshape(1, OC), mk)


def _run_blk_s2(x, w1, b1, wd, bd, w2, b2, H, W, G=1):
    N, Sp_in, C = x.shape
    Wp_in = W + 2
    H2, W2 = H // 2, W // 2
    W2p = W2 + 2
    M = H2 * W2p
    Sp = (H2 + 4) * W2p
    OC = w1.shape[-1]
    w1, w2 = _cat_w(w1), _cat_w(w2)
    mk1 = _col_mask(W2, W2p, M)
    mk2 = _interior_mask(H2, W2, W2p, M)
    return pl.pallas_call(
        functools.partial(_blk_s2_kernel, Wp_in=Wp_in, H=H, W=W, W2p=W2p,
                          M=M, Sp=Sp, G=G),
        out_shape=jax.ShapeDtypeStruct((N, Sp, OC), jnp.bfloat16),
        grid=(N // G,),
        in_specs=[pl.BlockSpec((G, Sp_in // 2, 2 * C), lambda n: (n, 0, 0)),
                  _wspec(w1),
                  pl.BlockSpec((1, OC), lambda n: (0, 0)),
                  pl.BlockSpec(wd.shape, lambda n: (0, 0)),
                  pl.BlockSpec((1, OC), lambda n: (0, 0)),
                  _wspec(w2),
                  pl.BlockSpec((1, OC), lambda n: (0, 0)),
                  pl.BlockSpec((M, 1), lambda n: (0, 0)),
                  pl.BlockSpec((M, 1), lambda n: (0, 0))],
        out_specs=pl.BlockSpec((G, Sp, OC), lambda n: (n, 0, 0)),
        scratch_shapes=[pltpu.VMEM((4, (H2 + 2) * W2p, C), jnp.bfloat16),
                        pltpu.VMEM((Sp, OC), jnp.bfloat16)],
        compiler_params=_cp(1),
    )(x.reshape(N, Sp_in // 2, 2 * C), w1, b1.reshape(1, OC),
      wd, bd.reshape(1, OC), w2, b2.reshape(1, OC), mk1, mk2)


def _run_head(x, w, b, cnt):
    N, Sp, C = x.shape
    D = w.shape[-1]
    return pl.pallas_call(
        functools.partial(_head_kernel, inv_cnt=1.0 / cnt),
        out_shape=jax.ShapeDtypeStruct((N, D), jnp.float32),
        grid=(1,),
        in_specs=[pl.BlockSpec((N, Sp, C), lambda i: (0, 0, 0)),
                  pl.BlockSpec((C, D), lambda i: (0, 0)),
                  pl.BlockSpec((1, D), lambda i: (0, 0))],
        out_specs=pl.BlockSpec((N, D), lambda i: (0, 0)),
        compiler_params=pltpu.CompilerParams(
            dimension_semantics=("arbitrary",),
            vmem_limit_bytes=_VMEM_LIMIT),
    )(x, w, b.reshape(1, D))


def kernel(img, conv1_w, conv1_b, fc_w, fc_b,
           l1_b0_c1_w, l1_b0_c1_b, l1_b0_c2_w, l1_b0_c2_b,
           l1_b1_c1_w, l1_b1_c1_b, l1_b1_c2_w, l1_b1_c2_b,
           l2_b0_c1_w, l2_b0_c1_b, l2_b0_c2_w, l2_b0_c2_b, l2_b0_ds_w, l2_b0_ds_b,
           l2_b1_c1_w, l2_b1_c1_b, l2_b1_c2_w, l2_b1_c2_b,
           l3_b0_c1_w, l3_b0_c1_b, l3_b0_c2_w, l3_b0_c2_b, l3_b0_ds_w, l3_b0_ds_b,
           l3_b1_c1_w, l3_b1_c1_b, l3_b1_c2_w, l3_b1_c2_b,
           l4_b0_c1_w, l4_b0_c1_b, l4_b0_c2_w, l4_b0_c2_b, l4_b0_ds_w, l4_b0_ds_b,
           l4_b1_c1_w, l4_b1_c1_b, l4_b1_c2_w, l4_b1_c2_b):
    N, _, HI, WI = img.shape
    oh, ow = HI // 2, WI // 2
    H1 = oh // 2
    x = jnp.transpose(img, (0, 2, 3, 1)).astype(jnp.bfloat16)

    xp = jnp.pad(x, ((0, 0), (3, 3), (3, 3), (0, 0)))
    cols = [xp[:, i:i + 2 * oh:2, j:j + 2 * ow:2, :]
            for i in range(7) for j in range(7)]
    patches = jnp.stack(cols, axis=3).reshape(N * oh * ow, 7 * 7 * 3)
    stem = _stem_matmul(patches, conv1_w, conv1_b).reshape(N, oh * ow, 64)

    x1 = _run_maxpool(stem, oh, ow)

    x1 = _run_blk_s1(x1, l1_b0_c1_w, l1_b0_c1_b, l1_b0_c2_w, l1_b0_c2_b, H1, H1)
    x1 = _run_blk_s1(x1, l1_b1_c1_w, l1_b1_c1_b, l1_b1_c2_w, l1_b1_c2_b, H1, H1)

    H2 = H1 // 2
    x2 = _run_blk_s2(x1, l2_b0_c1_w, l2_b0_c1_b, l2_b0_ds_w, l2_b0_ds_b,
                     l2_b0_c2_w, l2_b0_c2_b, H1, H1)
    x2 = _run_blk_s1(x2, l2_b1_c1_w, l2_b1_c1_b, l2_b1_c2_w, l2_b1_c2_b, H2, H2)

    H3 = H2 // 2
    x3 = _run_blk_s2(x2, l3_b0_c1_w, l3_b0_c1_b, l3_b0_ds_w, l3_b0_ds_b,
                     l3_b0_c2_w, l3_b0_c2_b, H2, H2)
    x3 = _run_blk_s1(x3, l3_b1_c1_w, l3_b1_c1_b, l3_b1_c2_w, l3_b1_c2_b, H3, H3)

    H4 = H3 // 2
    G4 = 4 if N % 4 == 0 else 1
    x4 = _run_blk_s2(x3, l4_b0_c1_w, l4_b0_c1_b, l4_b0_ds_w, l4_b0_ds_b,
                     l4_b0_c2_w, l4_b0_c2_b, H3, H3, G=G4)
    x4 = _run_blk_s1(x4, l4_b1_c1_w, l4_b1_c1_b, l4_b1_c2_w, l4_b1_c2_b, H4, H4,
                     G=G4)

    logits = _run_head(x4, fc_w, fc_b, float(H4 * H4))
    return logits, None, None
